# Initial kernel scaffold; baseline (speedup 1.0000x reference)
import jax
import jax.numpy as jnp
from jax.experimental import pallas as pl


def kernel(x, weights):
    raise NotImplementedError("write your pallas kernel here")


# speedup vs baseline: 2.5551x; 2.5551x over previous
import functools

import jax
import jax.numpy as jnp
from jax import lax
from jax.experimental import pallas as pl
from jax.experimental.pallas import tpu as pltpu
from jax.experimental.pallas import tpu_sc as plsc

B = 4096
M = 16
D = 64
N = M * M
NC = 2
NS = 16
NW = NC * NS
ROWS = B // NW
BLK = 512


def _scores_body(x_ref, wt_ref, s_ref):
    wt = wt_ref[...]
    w2 = jnp.sum(wt * wt, axis=0, keepdims=True)
    s_ref[...] = w2 - 2.0 * jnp.dot(
        x_ref[...], wt, preferred_element_type=jnp.float32,
        precision=lax.Precision.HIGHEST)


def _tc_scores(x, wt):
    return pl.pallas_call(
        _scores_body,
        grid=(B // BLK,),
        in_specs=[
            pl.BlockSpec((BLK, D), lambda i: (i, 0)),
            pl.BlockSpec((D, N), lambda i: (0, 0)),
        ],
        out_specs=pl.BlockSpec((BLK, N), lambda i: (i, 0)),
        out_shape=jax.ShapeDtypeStruct((B, N), jnp.float32),
    )(x, wt)


def _argmin_body(s_hbm, o_hbm, s_v, o_v):
    wid = lax.axis_index("s") * NC + lax.axis_index("c")
    base = wid * ROWS
    pltpu.sync_copy(s_hbm.at[pl.ds(base, ROWS)], s_v)

    def row(r, carry):
        best = s_v[r, pl.ds(0, M)]
        bidx = jnp.zeros((M,), jnp.int32)
        for k in range(1, M):
            v = s_v[r, pl.ds(k * M, M)]
            m = v < best
            best = jnp.where(m, v, best)
            bidx = jnp.where(m, jnp.int32(k), bidx)
        o_v[r, pl.ds(0, M)] = bidx
        return carry

    lax.fori_loop(0, ROWS, row, 0)
    pltpu.sync_copy(o_v, o_hbm.at[pl.ds(base, ROWS)])


@functools.cache
def _sc_argmin():
    return pl.kernel(
        _argmin_body,
        out_type=jax.ShapeDtypeStruct((B, M), jnp.int32),
        mesh=plsc.VectorSubcoreMesh(core_axis_name="c", subcore_axis_name="s"),
        scratch_types=[
            pltpu.VMEM((ROWS, N), jnp.float32),
            pltpu.VMEM((ROWS, M), jnp.int32),
        ],
    )


def kernel(x, weights):
    wt = weights.transpose(1, 0, 2).reshape(N, D).T
    scores = _tc_scores(x, wt)
    return _sc_argmin()(scores)
